# baseline (speedup 1.0000x reference)
import functools

import jax
import jax.numpy as jnp
from jax import lax
from jax.experimental import pallas as pl
from jax.experimental.pallas import tpu as pltpu

B = 10
N = 1000
NP = 1024
K = 16
C = 32
NCLS = 40
TOPK_CHUNK = 128
EDGE_CHUNK = 64
_BIG = 3.0e38


def _pointnet_body(posP, posT, wa1, wp1, b1a, w1b, b1b, wh2, wp2, b2a,
                   w2b, b2b, wc, bc, out, idxS, aS, bS, h1S, h2S):
    p = posP[0]
    pT = posT[0]
    sq_row = jnp.sum(pT * pT, axis=0, keepdims=True)
    col_i = lax.broadcasted_iota(jnp.int32, (TOPK_CHUNK, NP), 1)

    def topk_chunk(c, _):
        base = c * TOPK_CHUNK
        pc = posP[0, pl.ds(base, TOPK_CHUNK), :]
        sqc = jnp.sum(pc * pc, axis=1, keepdims=True)
        d = (sqc + sq_row) - 2.0 * jnp.dot(pc, pT)
        d = jnp.where(col_i >= N, _BIG, d)
        cols = []
        for _k in range(K):
            m = jnp.min(d, axis=1, keepdims=True)
            am = jnp.min(jnp.where(d == m, col_i, jnp.int32(1 << 30)),
                         axis=1, keepdims=True)
            cols.append(am)
            d = jnp.where(col_i == am, _BIG, d)
        idxS[pl.ds(base, TOPK_CHUNK), :] = jnp.concatenate(cols, axis=1)
        return 0

    for _c in range(NP // TOPK_CHUNK):
        topk_chunk(_c, 0)

    iota_src = lax.broadcasted_iota(jnp.int32, (EDGE_CHUNK, K, NP), 2)

    def layer(dst_ref, w2, b2row):
        def body(c, _):
            base = c * EDGE_CHUNK
            idxc = idxS[pl.ds(base, EDGE_CHUNK), :]
            oh = (idxc[:, :, None] == iota_src).astype(jnp.float32)
            g = jnp.dot(oh.reshape(EDGE_CHUNK * K, NP), aS[:, :])
            pre = (g.reshape(EDGE_CHUNK, K, C)
                   - bS[pl.ds(base, EDGE_CHUNK), :][:, None, :])
            act = jnp.maximum(pre, 0.0).reshape(EDGE_CHUNK * K, C)
            msg = jnp.dot(act, w2) + b2row
            mx = jnp.max(msg.reshape(EDGE_CHUNK, K, C), axis=1)
            dst_ref[pl.ds(base, EDGE_CHUNK), :] = jnp.maximum(mx, 0.0)
            return 0
        for _c in range(NP // EDGE_CHUNK):
            body(_c, 0)

    aS[:, :] = jnp.dot(p, wa1[:, :]) + b1a[0:1, :]
    bS[:, :] = jnp.dot(p, wp1[:, :])
    layer(h1S, w1b[:, :], b1b[0:1, :])

    p2 = jnp.dot(p, wp2[:, :])
    aS[:, :] = jnp.dot(h1S[:, :], wh2[:, :]) + p2 + b2a[0:1, :]
    bS[:, :] = p2
    layer(h2S, w2b[:, :], b2b[0:1, :])

    row_i = lax.broadcasted_iota(jnp.int32, (NP, C), 0)
    hm = jnp.where(row_i >= N, 0.0, h2S[:, :])
    gmax = jnp.max(hm, axis=0, keepdims=True)
    out[0] = jnp.dot(gmax, wc[:, :]) + bc[0:1, :]


def _w_specs():
    def full(shape):
        nd = len(shape)
        return pl.BlockSpec(shape, lambda g, _n=nd: (0,) * _n)
    return [
        pl.BlockSpec((1, NP, 8), lambda g: (g, 0, 0)),
        pl.BlockSpec((1, 8, NP), lambda g: (g, 0, 0)),
        full((8, C)),
        full((8, C)),
        full((8, C)),
        full((C, C)),
        full((8, C)),
        full((C, C)),
        full((8, C)),
        full((8, C)),
        full((C, C)),
        full((8, C)),
        full((C, NCLS)),
        full((8, NCLS)),
    ]


_IN_SPECS = _w_specs()
_OUT_SPEC = pl.BlockSpec((1, 1, NCLS), lambda g: (g, 0, 0))
_OUT_SHAPE = jax.ShapeDtypeStruct((B, 1, NCLS), jnp.float32)
_SCRATCH = [
    pltpu.VMEM((NP, K), jnp.int32),
    pltpu.VMEM((NP, C), jnp.float32),
    pltpu.VMEM((NP, C), jnp.float32),
    pltpu.VMEM((NP, C), jnp.float32),
    pltpu.VMEM((NP, C), jnp.float32),
]


def _padrow(x, rows=8):
    return jnp.zeros((rows, x.shape[-1]), x.dtype).at[: x.shape[0]].set(x)


@functools.partial(jax.jit, static_argnames=())
def kernel(pos, batch, W1a, b1a, W1b, b1b, W2a, b2a, W2b, b2b, Wc, bc):
    P = pos.reshape(B, N, 3)
    posP = jnp.zeros((B, NP, 8), jnp.float32).at[:, :N, :3].set(P)
    posT = posP.transpose(0, 2, 1)
    wa1 = _padrow(W1a[0:3] + W1a[3:6])
    wp1 = _padrow(W1a[3:6])
    wh2 = W2a[0:C]
    wp2 = _padrow(W2a[C:C + 3])
    args = (posP, posT, wa1, wp1, _padrow(b1a[None, :]), W1b,
            _padrow(b1b[None, :]), wh2, wp2, _padrow(b2a[None, :]), W2b,
            _padrow(b2b[None, :]), Wc, _padrow(bc[None, :]))
    out = pl.pallas_call(
        _pointnet_body,
        grid=(B,),
        in_specs=_IN_SPECS,
        out_specs=_OUT_SPEC,
        out_shape=_OUT_SHAPE,
        scratch_shapes=_SCRATCH,
        compiler_params=pltpu.CompilerParams(
            dimension_semantics=("arbitrary",)),
    )(*args)
    return out.reshape(B, NCLS)


# speedup vs baseline: 1.4539x; 1.4539x over previous
import functools

import jax
import jax.numpy as jnp
from jax import lax
from jax.experimental import pallas as pl
from jax.experimental.pallas import tpu as pltpu

B = 10
N = 1000
NP = 1024
K = 16
C = 32
NCLS = 40
TOPK_CHUNK = 128
EDGE_CHUNK = 64
_BIG = 3.0e38


def _pointnet_body(posP, posT, wa1, wp1, b1a, w1b, b1b, wh2, wp2, b2a,
                   w2b, b2b, wc, bc, out, idxS, aS, bS, h1S, h2S):
    p = posP[0]
    pT = posT[0]
    col_row = lax.broadcasted_iota(jnp.int32, (1, NP), 1)
    sq_row = (jnp.sum(pT * pT, axis=0, keepdims=True)
              + jnp.where(col_row >= N, _BIG, 0.0))
    col_i = lax.broadcasted_iota(jnp.int32, (TOPK_CHUNK, NP), 1)

    aS[:, :] = jnp.dot(p, wa1[:, :]) + b1a[0:1, :]
    bS[:, :] = jnp.dot(p, wp1[:, :])

    def topk_chunk(c, _):
        base = c * TOPK_CHUNK
        pc = posP[0, pl.ds(base, TOPK_CHUNK), :]
        sqc = jnp.sum(pc * pc, axis=1, keepdims=True)
        d = (sqc + sq_row) - 2.0 * jnp.dot(pc, pT)
        bc_ = bS[pl.ds(base, TOPK_CHUNK), :]
        cols = []
        mx = None
        for _k in range(K):
            m = jnp.min(d, axis=1, keepdims=True)
            z = d == m
            am = jnp.min(jnp.where(z, col_i, jnp.int32(1 << 30)),
                         axis=1, keepdims=True)
            cols.append(am)
            zf = z.astype(jnp.float32)
            d = d + zf * _BIG
            gk = jnp.dot(zf, aS[:, :])
            msg = jnp.dot(jnp.maximum(gk - bc_, 0.0), w1b[:, :])
            mx = msg if mx is None else jnp.maximum(mx, msg)
        idxS[pl.ds(base, TOPK_CHUNK), :] = jnp.concatenate(cols, axis=1)
        h1S[pl.ds(base, TOPK_CHUNK), :] = jnp.maximum(mx + b1b[0:1, :], 0.0)
        return 0

    lax.fori_loop(0, NP // TOPK_CHUNK, topk_chunk, 0, unroll=False)

    iota_src = lax.broadcasted_iota(jnp.int32, (EDGE_CHUNK, K, NP), 2)

    def layer(dst_ref, w2, b2row):
        def body(c, _):
            base = c * EDGE_CHUNK
            idxc = idxS[pl.ds(base, EDGE_CHUNK), :]
            oh = (idxc[:, :, None] == iota_src).astype(jnp.float32)
            g = jnp.dot(oh.reshape(EDGE_CHUNK * K, NP), aS[:, :])
            pre = (g.reshape(EDGE_CHUNK, K, C)
                   - bS[pl.ds(base, EDGE_CHUNK), :][:, None, :])
            act = jnp.maximum(pre, 0.0).reshape(EDGE_CHUNK * K, C)
            msg = jnp.dot(act, w2) + b2row
            mx = jnp.max(msg.reshape(EDGE_CHUNK, K, C), axis=1)
            dst_ref[pl.ds(base, EDGE_CHUNK), :] = jnp.maximum(mx, 0.0)
            return 0
        lax.fori_loop(0, NP // EDGE_CHUNK, body, 0, unroll=False)

    p2 = jnp.dot(p, wp2[:, :])
    aS[:, :] = jnp.dot(h1S[:, :], wh2[:, :]) + p2 + b2a[0:1, :]
    bS[:, :] = p2
    layer(h2S, w2b[:, :], b2b[0:1, :])

    row_i = lax.broadcasted_iota(jnp.int32, (NP, C), 0)
    hm = jnp.where(row_i >= N, 0.0, h2S[:, :])
    gmax = jnp.max(hm, axis=0, keepdims=True)
    out[0] = jnp.dot(gmax, wc[:, :]) + bc[0:1, :]


def _w_specs():
    def full(shape):
        nd = len(shape)
        return pl.BlockSpec(shape, lambda g, _n=nd: (0,) * _n)
    return [
        pl.BlockSpec((1, NP, 8), lambda g: (g, 0, 0)),
        pl.BlockSpec((1, 8, NP), lambda g: (g, 0, 0)),
        full((8, C)),
        full((8, C)),
        full((8, C)),
        full((C, C)),
        full((8, C)),
        full((C, C)),
        full((8, C)),
        full((8, C)),
        full((C, C)),
        full((8, C)),
        full((C, NCLS)),
        full((8, NCLS)),
    ]


_IN_SPECS = _w_specs()
_OUT_SPEC = pl.BlockSpec((1, 1, NCLS), lambda g: (g, 0, 0))
_OUT_SHAPE = jax.ShapeDtypeStruct((B, 1, NCLS), jnp.float32)
_SCRATCH = [
    pltpu.VMEM((NP, K), jnp.int32),
    pltpu.VMEM((NP, C), jnp.float32),
    pltpu.VMEM((NP, C), jnp.float32),
    pltpu.VMEM((NP, C), jnp.float32),
    pltpu.VMEM((NP, C), jnp.float32),
]


def _padrow(x, rows=8):
    return jnp.zeros((rows, x.shape[-1]), x.dtype).at[: x.shape[0]].set(x)


@functools.partial(jax.jit, static_argnames=())
def kernel(pos, batch, W1a, b1a, W1b, b1b, W2a, b2a, W2b, b2b, Wc, bc):
    P = pos.reshape(B, N, 3)
    posP = jnp.zeros((B, NP, 8), jnp.float32).at[:, :N, :3].set(P)
    posT = posP.transpose(0, 2, 1)
    wa1 = _padrow(W1a[0:3] + W1a[3:6])
    wp1 = _padrow(W1a[3:6])
    wh2 = W2a[0:C]
    wp2 = _padrow(W2a[C:C + 3])
    args = (posP, posT, wa1, wp1, _padrow(b1a[None, :]), W1b,
            _padrow(b1b[None, :]), wh2, wp2, _padrow(b2a[None, :]), W2b,
            _padrow(b2b[None, :]), Wc, _padrow(bc[None, :]))
    out = pl.pallas_call(
        _pointnet_body,
        grid=(B,),
        in_specs=_IN_SPECS,
        out_specs=_OUT_SPEC,
        out_shape=_OUT_SHAPE,
        scratch_shapes=_SCRATCH,
        compiler_params=pltpu.CompilerParams(
            dimension_semantics=("arbitrary",)),
    )(*args)
    return out.reshape(B, NCLS)


# speedup vs baseline: 1.5998x; 1.1003x over previous
import functools

import jax
import jax.numpy as jnp
from jax import lax
from jax.experimental import pallas as pl
from jax.experimental.pallas import tpu as pltpu

B = 10
N = 1000
NP = 1024
K = 16
C = 32
NCLS = 40
TOPK_CHUNK = 128
EDGE_CHUNK = 64
_BIG = 3.0e38


def _pointnet_body(posP, posT, wa1, wp1, b1a, w1b, b1b, wh2, wp2, b2a,
                   w2b, b2b, wc, bc, out, idxS, aS, bS, h1S, h2S):
    p = posP[0]
    pT = posT[0]
    col_row = lax.broadcasted_iota(jnp.int32, (1, NP), 1)
    sq_row = (jnp.sum(pT * pT, axis=0, keepdims=True)
              + jnp.where(col_row >= N, _BIG, 0.0))
    col_i = lax.broadcasted_iota(jnp.int32, (TOPK_CHUNK, NP), 1)

    aS[:, :] = jnp.dot(p, wa1[:, :]) + b1a[0:1, :]
    bS[:, :] = jnp.dot(p, wp1[:, :])

    def topk_chunk(c, _):
        base = c * TOPK_CHUNK
        pc = posP[0, pl.ds(base, TOPK_CHUNK), :]
        sqc = jnp.sum(pc * pc, axis=1, keepdims=True)
        d = (sqc + sq_row) - 2.0 * jnp.dot(pc, pT)
        bc_ = bS[pl.ds(base, TOPK_CHUNK), :]
        bits = lax.bitcast_convert_type(d, jnp.int32)
        s = bits ^ ((bits >> 31) & jnp.int32(0x7FFFFFFF))
        key = (s & jnp.int32(-1024)) | col_i
        cols = []
        mx = None
        for _k in range(K):
            kmin = jnp.min(key, axis=1, keepdims=True)
            z = key == kmin
            key = jnp.where(z, jnp.int32(0x7FFFFFFF), key)
            cols.append(kmin & jnp.int32(1023))
            zf = z.astype(jnp.float32)
            gk = jnp.dot(zf, aS[:, :])
            msg = jnp.dot(jnp.maximum(gk - bc_, 0.0), w1b[:, :])
            mx = msg if mx is None else jnp.maximum(mx, msg)
        idxS[pl.ds(base, TOPK_CHUNK), :] = jnp.concatenate(cols, axis=1)
        h1S[pl.ds(base, TOPK_CHUNK), :] = jnp.maximum(mx + b1b[0:1, :], 0.0)
        return 0

    lax.fori_loop(0, NP // TOPK_CHUNK, topk_chunk, 0, unroll=False)

    iota_src = lax.broadcasted_iota(jnp.int32, (EDGE_CHUNK, K, NP), 2)

    def layer(dst_ref, w2, b2row):
        def body(c, _):
            base = c * EDGE_CHUNK
            idxc = idxS[pl.ds(base, EDGE_CHUNK), :]
            oh = (idxc[:, :, None] == iota_src).astype(jnp.float32)
            g = jnp.dot(oh.reshape(EDGE_CHUNK * K, NP), aS[:, :])
            pre = (g.reshape(EDGE_CHUNK, K, C)
                   - bS[pl.ds(base, EDGE_CHUNK), :][:, None, :])
            act = jnp.maximum(pre, 0.0).reshape(EDGE_CHUNK * K, C)
            msg = jnp.dot(act, w2) + b2row
            mx = jnp.max(msg.reshape(EDGE_CHUNK, K, C), axis=1)
            dst_ref[pl.ds(base, EDGE_CHUNK), :] = jnp.maximum(mx, 0.0)
            return 0
        lax.fori_loop(0, NP // EDGE_CHUNK, body, 0, unroll=False)

    p2 = jnp.dot(p, wp2[:, :])
    aS[:, :] = jnp.dot(h1S[:, :], wh2[:, :]) + p2 + b2a[0:1, :]
    bS[:, :] = p2
    layer(h2S, w2b[:, :], b2b[0:1, :])

    row_i = lax.broadcasted_iota(jnp.int32, (NP, C), 0)
    hm = jnp.where(row_i >= N, 0.0, h2S[:, :])
    gmax = jnp.max(hm, axis=0, keepdims=True)
    out[0] = jnp.dot(gmax, wc[:, :]) + bc[0:1, :]


def _w_specs():
    def full(shape):
        nd = len(shape)
        return pl.BlockSpec(shape, lambda g, _n=nd: (0,) * _n)
    return [
        pl.BlockSpec((1, NP, 8), lambda g: (g, 0, 0)),
        pl.BlockSpec((1, 8, NP), lambda g: (g, 0, 0)),
        full((8, C)),
        full((8, C)),
        full((8, C)),
        full((C, C)),
        full((8, C)),
        full((C, C)),
        full((8, C)),
        full((8, C)),
        full((C, C)),
        full((8, C)),
        full((C, NCLS)),
        full((8, NCLS)),
    ]


_IN_SPECS = _w_specs()
_OUT_SPEC = pl.BlockSpec((1, 1, NCLS), lambda g: (g, 0, 0))
_OUT_SHAPE = jax.ShapeDtypeStruct((B, 1, NCLS), jnp.float32)
_SCRATCH = [
    pltpu.VMEM((NP, K), jnp.int32),
    pltpu.VMEM((NP, C), jnp.float32),
    pltpu.VMEM((NP, C), jnp.float32),
    pltpu.VMEM((NP, C), jnp.float32),
    pltpu.VMEM((NP, C), jnp.float32),
]


def _padrow(x, rows=8):
    return jnp.zeros((rows, x.shape[-1]), x.dtype).at[: x.shape[0]].set(x)


@functools.partial(jax.jit, static_argnames=())
def kernel(pos, batch, W1a, b1a, W1b, b1b, W2a, b2a, W2b, b2b, Wc, bc):
    P = pos.reshape(B, N, 3)
    posP = jnp.zeros((B, NP, 8), jnp.float32).at[:, :N, :3].set(P)
    posT = posP.transpose(0, 2, 1)
    wa1 = _padrow(W1a[0:3] + W1a[3:6])
    wp1 = _padrow(W1a[3:6])
    wh2 = W2a[0:C]
    wp2 = _padrow(W2a[C:C + 3])
    args = (posP, posT, wa1, wp1, _padrow(b1a[None, :]), W1b,
            _padrow(b1b[None, :]), wh2, wp2, _padrow(b2a[None, :]), W2b,
            _padrow(b2b[None, :]), Wc, _padrow(bc[None, :]))
    out = pl.pallas_call(
        _pointnet_body,
        grid=(B,),
        in_specs=_IN_SPECS,
        out_specs=_OUT_SPEC,
        out_shape=_OUT_SHAPE,
        scratch_shapes=_SCRATCH,
        compiler_params=pltpu.CompilerParams(
            dimension_semantics=("arbitrary",)),
    )(*args)
    return out.reshape(B, NCLS)
